# baseline (speedup 1.0000x reference)
import jax
import jax.numpy as jnp
from jax.experimental import pallas as pl
from jax.experimental.pallas import tpu as pltpu

TAU_ = 0.07
M_, B_, C_, H_, W_ = 5, 4, 64, 100, 250
N_ = M_ * B_
HW_ = H_ * W_
HSPLIT_ = 2


def _fused_kernel(q_ref, k_ref, out_ref, qacc, kacc):
    i = pl.program_id(0)
    j = pl.program_id(1)
    h = pl.program_id(2)
    t = i * B_ + j
    qp = jnp.sum(q_ref[...], axis=(2, 4)).reshape(1, C_)
    kp = jnp.sum(k_ref[...], axis=(2, 4)).reshape(1, C_)

    @pl.when(h == 0)
    def _():
        qacc[pl.ds(t, 1), :] = qp
        kacc[pl.ds(t, 1), :] = kp

    @pl.when(h != 0)
    def _():
        qacc[pl.ds(t, 1), :] = qacc[pl.ds(t, 1), :] + qp
        kacc[pl.ds(t, 1), :] = kacc[pl.ds(t, 1), :] + kp

    @pl.when(jnp.logical_and(i == M_ - 1,
                             jnp.logical_and(j == B_ - 1, h == HSPLIT_ - 1)))
    def _():
        inv = 1.0 / HW_
        qm = qacc[...] * inv
        km = kacc[...] * inv
        nq = qm / jnp.maximum(
            jnp.sqrt(jnp.sum(qm * qm, axis=1, keepdims=True)), 1e-12)
        nk = km / jnp.maximum(
            jnp.sqrt(jnp.sum(km * km, axis=1, keepdims=True)), 1e-12)
        sim = jax.lax.dot_general(
            nk, nq, (((1,), (1,)), ((), ())),
            preferred_element_type=jnp.float32)
        logits = sim * (1.0 / TAU_)
        mx = jnp.max(logits, axis=1, keepdims=True)
        lse = jnp.log(jnp.sum(jnp.exp(logits - mx), axis=1,
                              keepdims=True)) + mx
        row = jax.lax.broadcasted_iota(jnp.int32, (N_, N_), 0)
        col = jax.lax.broadcasted_iota(jnp.int32, (N_, N_), 1)
        diag = jnp.sum(jnp.where(row == col, logits, 0.0), axis=1,
                       keepdims=True)
        ce = lse - diag
        pad = (km[:, 0:1] != 0.0).astype(jnp.float32)
        num = jnp.sum(ce * pad, keepdims=True)
        den = jnp.maximum(jnp.sum(pad, keepdims=True), 1.0)
        out_ref[...] = num / den


def kernel(features_q, features_k, pos_region_ranges):
    del pos_region_ranges
    qt = jnp.transpose(features_q, (0, 1, 3, 2, 4))
    kt = jnp.transpose(features_k, (0, 1, 3, 2, 4))
    loss = pl.pallas_call(
        _fused_kernel,
        grid=(M_, B_, HSPLIT_),
        in_specs=[
            pl.BlockSpec((1, 1, H_ // HSPLIT_, C_, W_),
                         lambda i, j, h: (i, j, h, 0, 0)),
            pl.BlockSpec((1, 1, H_ // HSPLIT_, C_, W_),
                         lambda i, j, h: (i, j, h, 0, 0)),
        ],
        out_specs=pl.BlockSpec((1, 1), lambda i, j, h: (0, 0)),
        out_shape=jax.ShapeDtypeStruct((1, 1), jnp.float32),
        scratch_shapes=[
            pltpu.VMEM((N_, C_), jnp.float32),
            pltpu.VMEM((N_, C_), jnp.float32),
        ],
        compiler_params=pltpu.CompilerParams(
            dimension_semantics=("arbitrary", "arbitrary", "arbitrary")),
    )(qt, kt)
    return loss.reshape(())
